# baseline (speedup 1.0000x reference)
import functools

import jax
import jax.numpy as jnp
from jax import lax
from jax.experimental import pallas as pl
from jax.experimental.pallas import tpu as pltpu
from jax.experimental.pallas import tpu_sc as plsc

CHUNK = 128


@functools.lru_cache(maxsize=None)
def _build(seq, embed, vocab):
    info = plsc.get_sparse_core_info()
    nw = info.num_cores * info.num_subcores
    b_per_w = seq // nw
    n_chunks = b_per_w // CHUNK
    mesh = plsc.VectorSubcoreMesh(core_axis_name="c", subcore_axis_name="s")

    @functools.partial(
        pl.kernel,
        mesh=mesh,
        out_type=jax.ShapeDtypeStruct((seq, embed), jnp.float32),
        scratch_types=[
            pltpu.VMEM((n_chunks, CHUNK), jnp.int32),
            pltpu.VMEM((b_per_w, embed), jnp.float32),
            pltpu.SemaphoreType.DMA,
        ],
        compiler_params=pltpu.CompilerParams(use_tc_tiling_on_sc=False),
    )
    def emb(sentence_hbm, table_hbm, out_hbm, idx_v, rows_v, sem):
        wid = lax.axis_index("s") * info.num_cores + lax.axis_index("c")
        base = wid * b_per_w
        for j in range(n_chunks):
            pltpu.sync_copy(
                sentence_hbm.at[pl.ds(base + j * CHUNK, CHUNK)], idx_v.at[j]
            )
        copies = []
        for j in range(n_chunks):
            copies.append(
                pltpu.async_copy(
                    table_hbm.at[idx_v.at[j]],
                    rows_v.at[pl.ds(j * CHUNK, CHUNK)],
                    sem,
                )
            )
        for c in copies:
            c.wait()
        pltpu.sync_copy(rows_v, out_hbm.at[pl.ds(base, b_per_w)])

    return emb


def kernel(sentence, table):
    vocab, embed = table.shape
    emb = _build(sentence.shape[0], embed, vocab)
    table_lin = table * jnp.float32(1.0 + 2.0**-23)
    return emb(sentence, table_lin)


# speedup vs baseline: 5.8035x; 5.8035x over previous
import functools

import jax
import jax.numpy as jnp
from jax import lax
from jax.experimental import pallas as pl
from jax.experimental.pallas import tpu as pltpu
from jax.experimental.pallas import tpu_sc as plsc

L = 16


@functools.lru_cache(maxsize=None)
def _build(seq, embed, vocab):
    info = plsc.get_sparse_core_info()
    nw = info.num_cores * info.num_subcores
    b_per_w = seq // nw
    n_groups = b_per_w // L
    mesh = plsc.VectorSubcoreMesh(core_axis_name="c", subcore_axis_name="s")

    @functools.partial(
        pl.kernel,
        mesh=mesh,
        out_type=jax.ShapeDtypeStruct((embed, seq), jnp.float32),
        scratch_types=[
            pltpu.VMEM((b_per_w,), jnp.int32),
            pltpu.VMEM((L, embed, 128), jnp.float32),
            pltpu.VMEM((embed, b_per_w), jnp.float32),
            pltpu.SemaphoreType.DMA,
        ],
        compiler_params=pltpu.CompilerParams(needs_layout_passes=False),
    )
    def emb(sentence_hbm, table_t_hbm, out_t_hbm, sidx_v, blk_v, cols_v, sem):
        wid = lax.axis_index("s") * info.num_cores + lax.axis_index("c")
        base = wid * b_per_w
        pltpu.sync_copy(sentence_hbm.at[pl.ds(base, b_per_w)], sidx_v)

        rows_lo = lax.iota(jnp.int32, L)
        rows_hi = rows_lo + L

        def group(g, _):
            iv = sidx_v[pl.ds(g * L, L)]
            copies = []
            for k in range(L):
                i = iv[k]
                tile_col = pl.multiple_of((i // 128) * 128, 128)
                copies.append(
                    pltpu.async_copy(
                        table_t_hbm.at[:, pl.ds(tile_col, 128)],
                        blk_v.at[k],
                        sem,
                    )
                )
            for c in copies:
                c.wait()
            for k in range(L):
                i = iv[k]
                c = jnp.full((L,), i % 128, jnp.int32)
                lo = plsc.load_gather(blk_v.at[k], [rows_lo, c])
                hi = plsc.load_gather(blk_v.at[k], [rows_hi, c])
                jcol = jnp.full((L,), g * L + k, jnp.int32)
                plsc.store_scatter(cols_v, [rows_lo, jcol], lo)
                plsc.store_scatter(cols_v, [rows_hi, jcol], hi)
            return 0

        lax.fori_loop(0, n_groups, group, 0)
        pltpu.sync_copy(cols_v, out_t_hbm.at[:, pl.ds(base, b_per_w)])

    return emb


def kernel(sentence, table):
    vocab, embed = table.shape
    emb = _build(sentence.shape[0], embed, vocab)
    out_t = emb(sentence, table.T)
    return out_t.T


# speedup vs baseline: 6.1761x; 1.0642x over previous
import functools

import jax
import jax.numpy as jnp
from jax import lax
from jax.experimental import pallas as pl
from jax.experimental.pallas import tpu as pltpu
from jax.experimental.pallas import tpu_sc as plsc

L = 16
G = 8
NBUF = 2


@functools.lru_cache(maxsize=None)
def _build(seq, embed, vocab):
    info = plsc.get_sparse_core_info()
    nw = info.num_cores * info.num_subcores
    b_per_w = seq // nw
    n_groups = b_per_w // G
    mesh = plsc.VectorSubcoreMesh(core_axis_name="c", subcore_axis_name="s")

    @functools.partial(
        pl.kernel,
        mesh=mesh,
        out_type=jax.ShapeDtypeStruct((embed, seq), jnp.float32),
        scratch_types=[
            pltpu.VMEM((b_per_w + L,), jnp.int32),
            pltpu.VMEM((NBUF, G, embed, 128), jnp.float32),
            pltpu.VMEM((embed, b_per_w), jnp.float32),
            pltpu.SemaphoreType.DMA,
            pltpu.SemaphoreType.DMA,
        ],
        compiler_params=pltpu.CompilerParams(needs_layout_passes=False),
    )
    def emb(
        sentence_hbm, table_t_hbm, out_t_hbm, sidx_v, blk_v, cols_v, sem0, sem1
    ):
        wid = lax.axis_index("s") * info.num_cores + lax.axis_index("c")
        base = wid * b_per_w
        pltpu.sync_copy(
            sentence_hbm.at[pl.ds(base, b_per_w)],
            sidx_v.at[pl.ds(0, b_per_w)],
        )

        sems = (sem0, sem1)
        rows_lo = lax.iota(jnp.int32, L)

        def fire(g, b):
            iv = sidx_v[pl.ds(g * G, L)]
            for k in range(G):
                i = iv[k]
                tile_col = pl.multiple_of((i // 128) * 128, 128)
                pltpu.async_copy(
                    table_t_hbm.at[:, pl.ds(tile_col, 128)],
                    blk_v.at[b, k],
                    sems[b],
                )

        def drain_extract(g, b):
            for k in range(G):
                pltpu.make_async_copy(
                    table_t_hbm.at[:, pl.ds(0, 128)], blk_v.at[b, k], sems[b]
                ).wait()
            iv = sidx_v[pl.ds(g * G, L)]
            for k in range(G):
                i = iv[k]
                c = jnp.full((L,), i % 128, jnp.int32)
                jcol = jnp.full((L,), g * G + k, jnp.int32)
                blk = blk_v.at[b, k]
                for h in range(embed // L):
                    rows = rows_lo + h * L
                    vals = plsc.load_gather(blk, [rows, c])
                    plsc.store_scatter(cols_v, [rows, jcol], vals)

        for b in range(NBUF):
            fire(b, b)

        def body(q, _):
            for b in range(NBUF):
                g = q * NBUF + b
                drain_extract(g, b)
                gn = g + NBUF

                @pl.when(gn < n_groups)
                def _():
                    fire(gn, b)

            return 0

        lax.fori_loop(0, n_groups // NBUF, body, 0)
        pltpu.sync_copy(cols_v, out_t_hbm.at[:, pl.ds(base, b_per_w)])

    return emb


def kernel(sentence, table):
    vocab, embed = table.shape
    emb = _build(sentence.shape[0], embed, vocab)
    out_t = emb(sentence, table.T)
    return out_t.T


# speedup vs baseline: 6.7036x; 1.0854x over previous
import functools

import jax
import jax.numpy as jnp
from jax import lax
from jax.experimental import pallas as pl
from jax.experimental.pallas import tpu as pltpu
from jax.experimental.pallas import tpu_sc as plsc

L = 16
G = 8
NBUF = 3


@functools.lru_cache(maxsize=None)
def _build(seq, embed, vocab):
    info = plsc.get_sparse_core_info()
    nw = info.num_cores * info.num_subcores
    b_per_w = seq // nw
    n_groups = b_per_w // G
    mesh = plsc.VectorSubcoreMesh(core_axis_name="c", subcore_axis_name="s")

    @functools.partial(
        pl.kernel,
        mesh=mesh,
        out_type=jax.ShapeDtypeStruct((embed, seq), jnp.float32),
        scratch_types=[
            pltpu.VMEM((b_per_w + L,), jnp.int32),
            pltpu.VMEM((NBUF, G, embed, 128), jnp.float32),
            pltpu.VMEM((embed, b_per_w), jnp.float32),
            pltpu.SemaphoreType.DMA,
            pltpu.SemaphoreType.DMA,
            pltpu.SemaphoreType.DMA,
        ],
        compiler_params=pltpu.CompilerParams(needs_layout_passes=False),
    )
    def emb(
        sentence_hbm,
        table_t_hbm,
        out_t_hbm,
        sidx_v,
        blk_v,
        cols_v,
        sem0,
        sem1,
        sem2,
    ):
        wid = lax.axis_index("s") * info.num_cores + lax.axis_index("c")
        base = wid * b_per_w
        pltpu.sync_copy(
            sentence_hbm.at[pl.ds(base, b_per_w)],
            sidx_v.at[pl.ds(0, b_per_w)],
        )

        sems = (sem0, sem1, sem2)
        rows_lo = lax.iota(jnp.int32, L)

        def fire(g, b):
            iv = sidx_v[pl.ds(g * G, L)]
            for k in range(G):
                i = iv[k]
                tile_col = pl.multiple_of((i // 128) * 128, 128)
                pltpu.async_copy(
                    table_t_hbm.at[:, pl.ds(tile_col, 128)],
                    blk_v.at[b, k],
                    sems[b],
                )

        def drain_extract(g, b):
            for k in range(G):
                pltpu.make_async_copy(
                    table_t_hbm.at[:, pl.ds(0, 128)], blk_v.at[b, k], sems[b]
                ).wait()
            iv = sidx_v[pl.ds(g * G, L)]
            for k in range(G):
                i = iv[k]
                c = jnp.full((L,), i % 128, jnp.int32)
                jcol = jnp.full((L,), g * G + k, jnp.int32)
                blk = blk_v.at[b, k]
                for h in range(embed // L):
                    rows = rows_lo + h * L
                    vals = plsc.load_gather(blk, [rows, c])
                    plsc.store_scatter(cols_v, [rows, jcol], vals)

        for b in range(NBUF):
            fire(b, b)

        def body(q, _):
            for b in range(NBUF):
                g = q * NBUF + b
                drain_extract(g, b)
                gn = g + NBUF

                @pl.when(gn < n_groups)
                def _():
                    fire(gn, b)

            return 0

        n_main = (n_groups // NBUF) * NBUF
        lax.fori_loop(0, n_groups // NBUF, body, 0)
        for g in range(n_main, n_groups):
            drain_extract(g, g % NBUF)
        pltpu.sync_copy(cols_v, out_t_hbm.at[:, pl.ds(base, b_per_w)])

    return emb


def kernel(sentence, table):
    vocab, embed = table.shape
    emb = _build(sentence.shape[0], embed, vocab)
    out_t = emb(sentence, table.T)
    return out_t.T


# speedup vs baseline: 7.3435x; 1.0955x over previous
import functools

import jax
import jax.numpy as jnp
from jax import lax
from jax.experimental import pallas as pl
from jax.experimental.pallas import tpu as pltpu
from jax.experimental.pallas import tpu_sc as plsc

L = 16
G = 4
NBUF = 6


@functools.lru_cache(maxsize=None)
def _build(seq, embed, vocab):
    info = plsc.get_sparse_core_info()
    nw = info.num_cores * info.num_subcores
    b_per_w = seq // nw
    n_groups = b_per_w // G
    mesh = plsc.VectorSubcoreMesh(core_axis_name="c", subcore_axis_name="s")

    @functools.partial(
        pl.kernel,
        mesh=mesh,
        out_type=jax.ShapeDtypeStruct((embed, seq), jnp.float32),
        scratch_types=[
            pltpu.VMEM((b_per_w + L,), jnp.int32),
            pltpu.VMEM((NBUF, G, embed, 128), jnp.float32),
            pltpu.VMEM((embed, b_per_w), jnp.float32),
        ] + [pltpu.SemaphoreType.DMA] * NBUF,
        compiler_params=pltpu.CompilerParams(needs_layout_passes=False),
    )
    def emb(
        sentence_hbm,
        table_t_hbm,
        out_t_hbm,
        sidx_v,
        blk_v,
        cols_v,
        *sems,
    ):
        wid = lax.axis_index("s") * info.num_cores + lax.axis_index("c")
        base = wid * b_per_w
        pltpu.sync_copy(
            sentence_hbm.at[pl.ds(base, b_per_w)],
            sidx_v.at[pl.ds(0, b_per_w)],
        )

        rows_lo = lax.iota(jnp.int32, L)

        def fire(g, b):
            iv = sidx_v[pl.ds(g * G, L)]
            for k in range(G):
                i = iv[k]
                tile_col = pl.multiple_of((i // 128) * 128, 128)
                pltpu.async_copy(
                    table_t_hbm.at[:, pl.ds(tile_col, 128)],
                    blk_v.at[b, k],
                    sems[b],
                )

        def drain_extract(g, b):
            for k in range(G):
                pltpu.make_async_copy(
                    table_t_hbm.at[:, pl.ds(0, 128)], blk_v.at[b, k], sems[b]
                ).wait()
            iv = sidx_v[pl.ds(g * G, L)]
            for k in range(G):
                i = iv[k]
                c = jnp.full((L,), i % 128, jnp.int32)
                jcol = jnp.full((L,), g * G + k, jnp.int32)
                blk = blk_v.at[b, k]
                for h in range(embed // L):
                    rows = rows_lo + h * L
                    vals = plsc.load_gather(blk, [rows, c])
                    plsc.store_scatter(cols_v, [rows, jcol], vals)

        for b in range(NBUF):
            fire(b, b)

        def body(q, _):
            for b in range(NBUF):
                g = q * NBUF + b
                drain_extract(g, b)
                gn = g + NBUF

                @pl.when(gn < n_groups)
                def _():
                    fire(gn, b)

            return 0

        n_main = (n_groups // NBUF) * NBUF
        lax.fori_loop(0, n_groups // NBUF, body, 0)
        for g in range(n_main, n_groups):
            drain_extract(g, g % NBUF)
        pltpu.sync_copy(cols_v, out_t_hbm.at[:, pl.ds(base, b_per_w)])

    return emb


def kernel(sentence, table):
    vocab, embed = table.shape
    emb = _build(sentence.shape[0], embed, vocab)
    out_t = emb(sentence, table.T)
    return out_t.T


# speedup vs baseline: 7.3635x; 1.0027x over previous
import functools

import jax
import jax.numpy as jnp
from jax import lax
from jax.experimental import pallas as pl
from jax.experimental.pallas import tpu as pltpu
from jax.experimental.pallas import tpu_sc as plsc

L = 16
G = 4
NBUF = 6


@functools.lru_cache(maxsize=None)
def _build(seq, embed, vocab):
    info = plsc.get_sparse_core_info()
    nw = info.num_cores * info.num_subcores
    b_per_w = seq // nw
    n_groups = b_per_w // G
    mesh = plsc.VectorSubcoreMesh(core_axis_name="c", subcore_axis_name="s")

    @functools.partial(
        pl.kernel,
        mesh=mesh,
        out_type=jax.ShapeDtypeStruct((embed, seq), jnp.float32),
        scratch_types=[
            pltpu.VMEM((b_per_w + L,), jnp.int32),
            pltpu.VMEM((NBUF, G, embed, 128), jnp.float32),
            pltpu.VMEM((embed, b_per_w), jnp.float32),
        ] + [pltpu.SemaphoreType.DMA] * NBUF,
        compiler_params=pltpu.CompilerParams(needs_layout_passes=False),
    )
    def emb(
        sentence_hbm,
        table_t_hbm,
        out_t_hbm,
        sidx_v,
        blk_v,
        cols_v,
        *sems,
    ):
        wid = lax.axis_index("s") * info.num_cores + lax.axis_index("c")
        base = wid * b_per_w
        pltpu.sync_copy(
            sentence_hbm.at[pl.ds(base, b_per_w)],
            sidx_v.at[pl.ds(0, b_per_w)],
        )

        rows_lo = lax.iota(jnp.int32, L)

        def fire(g, b, lane0):
            iv = sidx_v[pl.ds((g // 2) * 8, L)]
            for k in range(G):
                i = iv[lane0 + k]
                tile_col = pl.multiple_of((i // 128) * 128, 128)
                pltpu.async_copy(
                    table_t_hbm.at[:, pl.ds(tile_col, 128)],
                    blk_v.at[b, k],
                    sems[b],
                )

        def drain_extract(g, b, lane0):
            for k in range(G):
                pltpu.make_async_copy(
                    table_t_hbm.at[:, pl.ds(0, 128)], blk_v.at[b, k], sems[b]
                ).wait()
            iv = sidx_v[pl.ds((g // 2) * 8, L)]
            for k in range(G):
                i = iv[lane0 + k]
                c = jnp.full((L,), i % 128, jnp.int32)
                jcol = jnp.full((L,), g * G + k, jnp.int32)
                blk = blk_v.at[b, k]
                for h in range(embed // L):
                    rows = rows_lo + h * L
                    vals = plsc.load_gather(blk, [rows, c])
                    plsc.store_scatter(cols_v, [rows, jcol], vals)

        for b in range(NBUF):
            fire(b, b, G * (b % 2))

        def body(q, _):
            for b in range(NBUF):
                g = q * NBUF + b
                lane0 = G * (b % 2)
                drain_extract(g, b, lane0)
                gn = g + NBUF

                @pl.when(gn < n_groups)
                def _():
                    fire(gn, b, lane0)

            return 0

        n_main = (n_groups // NBUF) * NBUF
        lax.fori_loop(0, n_groups // NBUF, body, 0)
        for g in range(n_main, n_groups):
            drain_extract(g, g % NBUF, G * (g % 2))
        pltpu.sync_copy(cols_v, out_t_hbm.at[:, pl.ds(base, b_per_w)])

    return emb


def kernel(sentence, table):
    vocab, embed = table.shape
    emb = _build(sentence.shape[0], embed, vocab)
    out_t = emb(sentence, table.T)
    return out_t.T
